# baseline (speedup 1.0000x reference)
import functools

import jax
import jax.numpy as jnp
from jax.experimental import pallas as pl
from jax.experimental.pallas import tpu as pltpu

BLK = 512


def _router_kernel(x_ref, wt_ref, b_ref, sel_ref, mask_ref):
    x = x_ref[...]
    wt = wt_ref[...]
    logits = jnp.dot(x, wt, preferred_element_type=jnp.float32)
    logits = logits + b_ref[...]
    sel = jnp.argmax(logits, axis=-1).astype(jnp.int32)
    P = logits.shape[-1]
    lane = jax.lax.broadcasted_iota(jnp.int32, logits.shape, 1)
    mask_ref[...] = (lane == sel[:, None]).astype(jnp.float32)
    sel_ref[0, 0, :] = sel


@functools.partial(jax.jit, static_argnames=())
def kernel(process_feats, routing_matrix, bias):
    B, N, D = process_feats.shape
    P = routing_matrix.shape[0]
    T = B * N
    x = process_feats.reshape(T, D)
    wt = routing_matrix.T
    b = bias.reshape(1, P)
    grid = (T // BLK,)
    sel2d, mask = pl.pallas_call(
        _router_kernel,
        grid=grid,
        in_specs=[
            pl.BlockSpec((BLK, D), lambda i: (i, 0)),
            pl.BlockSpec((D, P), lambda i: (0, 0)),
            pl.BlockSpec((1, P), lambda i: (0, 0)),
        ],
        out_specs=[
            pl.BlockSpec((1, 1, BLK), lambda i: (i, 0, 0)),
            pl.BlockSpec((BLK, P), lambda i: (i, 0)),
        ],
        out_shape=[
            jax.ShapeDtypeStruct((T // BLK, 1, BLK), jnp.int32),
            jax.ShapeDtypeStruct((T, P), jnp.float32),
        ],
        compiler_params=pltpu.CompilerParams(
            dimension_semantics=("parallel",),
        ),
    )(x, wt, b)
    selected = sel2d.reshape(B, N)
    policy_mask = mask.reshape(B, N, P)
    return (selected, policy_mask)


# speedup vs baseline: 1.1304x; 1.1304x over previous
import functools

import jax
import jax.numpy as jnp
from jax.experimental import pallas as pl
from jax.experimental.pallas import tpu as pltpu

BLK = 2048


def _router_kernel(x_ref, wt_ref, b_ref, sel_ref, mask_ref):
    x = x_ref[...]
    wt = wt_ref[...]
    logits = jnp.dot(x, wt, preferred_element_type=jnp.float32)
    logits = logits + b_ref[...]
    sel = jnp.argmax(logits, axis=-1).astype(jnp.int32)
    P = logits.shape[-1]
    lane = jax.lax.broadcasted_iota(jnp.int32, logits.shape, 1)
    mask_ref[...] = (lane == sel[:, None]).astype(jnp.float32)
    sel_ref[0, 0, :] = sel


@functools.partial(jax.jit, static_argnames=())
def kernel(process_feats, routing_matrix, bias):
    B, N, D = process_feats.shape
    P = routing_matrix.shape[0]
    T = B * N
    x = process_feats.reshape(T, D)
    wt = routing_matrix.T
    b = bias.reshape(1, P)
    grid = (T // BLK,)
    sel2d, mask = pl.pallas_call(
        _router_kernel,
        grid=grid,
        in_specs=[
            pl.BlockSpec((BLK, D), lambda i: (i, 0)),
            pl.BlockSpec((D, P), lambda i: (0, 0)),
            pl.BlockSpec((1, P), lambda i: (0, 0)),
        ],
        out_specs=[
            pl.BlockSpec((1, 1, BLK), lambda i: (i, 0, 0)),
            pl.BlockSpec((BLK, P), lambda i: (i, 0)),
        ],
        out_shape=[
            jax.ShapeDtypeStruct((T // BLK, 1, BLK), jnp.int32),
            jax.ShapeDtypeStruct((T, P), jnp.float32),
        ],
        compiler_params=pltpu.CompilerParams(
            dimension_semantics=("parallel",),
        ),
    )(x, wt, b)
    selected = sel2d.reshape(B, N)
    policy_mask = mask.reshape(B, N, P)
    return (selected, policy_mask)


# speedup vs baseline: 1.2637x; 1.1179x over previous
import functools

import jax
import jax.numpy as jnp
from jax.experimental import pallas as pl
from jax.experimental.pallas import tpu as pltpu

BLK = 2048


def _router_kernel(x_ref, w_ref, b_ref, sel_ref, mask_ref):
    x = x_ref[...]
    w = w_ref[...]
    P = w.shape[0]
    logits_t = jax.lax.dot_general(
        w, x, (((1,), (1,)), ((), ())),
        preferred_element_type=jnp.float32)
    logits_t = logits_t + b_ref[...]
    m = jnp.max(logits_t, axis=0, keepdims=True)
    sub = jax.lax.broadcasted_iota(jnp.int32, logits_t.shape, 0)
    sel = jnp.min(jnp.where(logits_t == m, sub, P), axis=0)
    sel = sel.astype(jnp.int32)
    mask_t = (sub == sel[None, :]).astype(jnp.float32)
    mask_ref[...] = mask_t.T
    sel_ref[0, 0, :] = sel


@functools.partial(jax.jit, static_argnames=())
def kernel(process_feats, routing_matrix, bias):
    B, N, D = process_feats.shape
    P = routing_matrix.shape[0]
    T = B * N
    x = process_feats.reshape(T, D)
    b = bias.reshape(P, 1)
    grid = (T // BLK,)
    sel2d, mask = pl.pallas_call(
        _router_kernel,
        grid=grid,
        in_specs=[
            pl.BlockSpec((BLK, D), lambda i: (i, 0)),
            pl.BlockSpec((P, D), lambda i: (0, 0)),
            pl.BlockSpec((P, 1), lambda i: (0, 0)),
        ],
        out_specs=[
            pl.BlockSpec((1, 1, BLK), lambda i: (i, 0, 0)),
            pl.BlockSpec((BLK, P), lambda i: (i, 0)),
        ],
        out_shape=[
            jax.ShapeDtypeStruct((T // BLK, 1, BLK), jnp.int32),
            jax.ShapeDtypeStruct((T, P), jnp.float32),
        ],
        compiler_params=pltpu.CompilerParams(
            dimension_semantics=("parallel",),
        ),
    )(x, routing_matrix, b)
    selected = sel2d.reshape(B, N)
    policy_mask = mask.reshape(B, N, P)
    return (selected, policy_mask)


# speedup vs baseline: 1.3145x; 1.0402x over previous
import functools

import jax
import jax.numpy as jnp
from jax.experimental import pallas as pl
from jax.experimental.pallas import tpu as pltpu

BLK = 1024


def _router_kernel(x_ref, w_ref, b_ref, sel_ref, mask_ref):
    x = x_ref[...]
    w = w_ref[...]
    P = w.shape[0]
    logits_t = jax.lax.dot_general(
        w, x, (((1,), (1,)), ((), ())),
        preferred_element_type=jnp.float32)
    logits_t = logits_t + b_ref[...]
    m = jnp.max(logits_t, axis=0, keepdims=True)
    sub = jax.lax.broadcasted_iota(jnp.int32, logits_t.shape, 0)
    sel = jnp.min(jnp.where(logits_t == m, sub, P), axis=0)
    sel = sel.astype(jnp.int32)
    mask_t = (sub == sel[None, :]).astype(jnp.float32)
    mask_ref[...] = mask_t.T
    sel_ref[0, 0, :] = sel


@functools.partial(jax.jit, static_argnames=())
def kernel(process_feats, routing_matrix, bias):
    B, N, D = process_feats.shape
    P = routing_matrix.shape[0]
    T = B * N
    x = process_feats.reshape(T, D)
    b = bias.reshape(P, 1)
    grid = (T // BLK,)
    sel2d, mask = pl.pallas_call(
        _router_kernel,
        grid=grid,
        in_specs=[
            pl.BlockSpec((BLK, D), lambda i: (i, 0)),
            pl.BlockSpec((P, D), lambda i: (0, 0)),
            pl.BlockSpec((P, 1), lambda i: (0, 0)),
        ],
        out_specs=[
            pl.BlockSpec((1, 1, BLK), lambda i: (i, 0, 0)),
            pl.BlockSpec((BLK, P), lambda i: (i, 0)),
        ],
        out_shape=[
            jax.ShapeDtypeStruct((T // BLK, 1, BLK), jnp.int32),
            jax.ShapeDtypeStruct((T, P), jnp.float32),
        ],
        compiler_params=pltpu.CompilerParams(
            dimension_semantics=("parallel",),
        ),
    )(x, routing_matrix, b)
    selected = sel2d.reshape(B, N)
    policy_mask = mask.reshape(B, N, P)
    return (selected, policy_mask)
